# baseline (speedup 1.0000x reference)
import functools

import jax
import jax.numpy as jnp
from jax import lax
from jax.experimental import pallas as pl
from jax.experimental.pallas import tpu as pltpu
from jax.experimental.pallas import tpu_sc as plsc

N = 10000
E = 320000
D = 128
V = 100
VP = 112
G = 400
NC = 2
NS = 16
NW = NC * NS
RPW = 312
RC = 104
EPW = E // NW
EC = 80
NCH = EPW // EC

_mesh = plsc.VectorSubcoreMesh(core_axis_name="c", subcore_axis_name="s")
_f32 = jnp.float32
_i32 = jnp.int32


EPB0 = 9984


def _prep_body(x_hbm, ea_hbm, dst_hbm, gp_hbm, embd_hbm,
               h0_hbm, ew0_hbm, ew1_hbm, ew2_hbm,
               embd_sh, gpv, gtab, xv0, xv1, xv2, hrows0, hrows1, hrows2,
               xv16, hrows16,
               didx_all, av_all, xdv_all, ewb0, ewb1, ewb2,
               sem0, sem1, sem2, sem3):
    c = lax.axis_index("c")
    s = lax.axis_index("s")
    w = c * NS + s
    ew_hbms = (ew0_hbm, ew1_hbm, ew2_hbm)
    ewbs = (ewb0, ewb1, ewb2)
    ebase = w * EPB0

    dd = pltpu.async_copy(dst_hbm.at[pl.ds(ebase, EPB0)], didx_all, sem0)
    de = pltpu.async_copy(ea_hbm.at[pl.ds(ebase, EPB0)], av_all, sem1)

    @pl.when(s == 0)
    def _():
        pltpu.sync_copy(embd_hbm, embd_sh)

    pltpu.sync_copy(gp_hbm, gpv)
    for i in range(3 * VP // 16):
        v = gpv[pl.ds(16 * i, 16)]
        gtab[pl.ds(16 * i, 16)] = 1.0 / (1.0 + jnp.exp(-v))
    plsc.subcore_barrier()

    rbase = w * RPW
    hrows = (hrows0, hrows1, hrows2)
    xvs = (xv0, xv1, xv2)
    xls = [pltpu.async_copy(x_hbm.at[pl.ds(rbase + j * RC, RC)], xvs[j],
                            sem2)
           for j in range(3)]
    for xl in xls:
        xl.wait()
    gds = [pltpu.async_copy(embd_sh.at[xvs[j]], hrows[j], sem2)
           for j in range(3)]
    for g in gds:
        g.wait()
    hds = [pltpu.async_copy(hrows[j], h0_hbm.at[pl.ds(rbase + j * RC, RC)],
                            sem2)
           for j in range(3)]
    for h in hds:
        h.wait()

    @pl.when(w == 0)
    def _():
        b = NW * RPW
        pltpu.sync_copy(x_hbm.at[pl.ds(b, 16)], xv16)
        pltpu.sync_copy(embd_sh.at[xv16], hrows16)
        pltpu.sync_copy(hrows16, h0_hbm.at[pl.ds(b, 16)])

    dd.wait()
    xds = [pltpu.async_copy(
        x_hbm.at[didx_all.at[pl.ds(j * 128, 128)]],
        xdv_all.at[pl.ds(j * 128, 128)], sem3)
        for j in range(EPB0 // 128)]
    for xd in xds:
        xd.wait()
    de.wait()

    def grp(g, _):
        a = av_all[pl.ds(g * 16, 16)]
        xi = xdv_all[pl.ds(g * 16, 16)]
        a2 = a * a
        for l in range(3):
            gl = plsc.load_gather(gtab, [xi + (l * VP)])
            ewbs[l][pl.ds(g * 16, 16)] = jnp.exp(-gl * a2)
        return 0

    lax.fori_loop(0, EPB0 // 16, grp, 0)
    eds = [pltpu.async_copy(ewbs[l], ew_hbms[l].at[pl.ds(ebase, EPB0)],
                            sem2)
           for l in range(3)]
    for e in eds:
        e.wait()

    @pl.when(w < 4)
    def _():
        bt = NW * EPB0 + w * 128
        pltpu.sync_copy(dst_hbm.at[pl.ds(bt, 128)],
                        didx_all.at[pl.ds(0, 128)])
        pltpu.sync_copy(ea_hbm.at[pl.ds(bt, 128)],
                        av_all.at[pl.ds(0, 128)])
        pltpu.sync_copy(x_hbm.at[didx_all.at[pl.ds(0, 128)]],
                        xdv_all.at[pl.ds(0, 128)])
        lax.fori_loop(0, 8, grp, 0)
        for l in range(3):
            pltpu.sync_copy(ewbs[l].at[pl.ds(0, 128)],
                            ew_hbms[l].at[pl.ds(bt, 128)])


_prep = pl.kernel(
    _prep_body,
    out_type=(jax.ShapeDtypeStruct((N, D), _f32),
              jax.ShapeDtypeStruct((E,), _f32),
              jax.ShapeDtypeStruct((E,), _f32),
              jax.ShapeDtypeStruct((E,), _f32)),
    mesh=_mesh,
    compiler_params=pltpu.CompilerParams(needs_layout_passes=False),
    scratch_types=[
        pltpu.VMEM_SHARED((V, D), _f32),
        pltpu.VMEM((3 * VP,), _f32),
        pltpu.VMEM((3 * VP,), _f32),
        pltpu.VMEM((RC,), _i32),
        pltpu.VMEM((RC,), _i32),
        pltpu.VMEM((RC,), _i32),
        pltpu.VMEM((RC, D), _f32),
        pltpu.VMEM((RC, D), _f32),
        pltpu.VMEM((RC, D), _f32),
        pltpu.VMEM((16,), _i32),
        pltpu.VMEM((16, D), _f32),
        pltpu.VMEM((EPB0,), _i32),
        pltpu.VMEM((EPB0,), _f32),
        pltpu.VMEM((EPB0,), _i32),
        pltpu.VMEM((EPB0,), _f32),
        pltpu.VMEM((EPB0,), _f32),
        pltpu.VMEM((EPB0,), _f32),
        pltpu.SemaphoreType.DMA,
        pltpu.SemaphoreType.DMA,
        pltpu.SemaphoreType.DMA,
        pltpu.SemaphoreType.DMA,
    ],
)


CB = 128
CPW = 78
EPB = CPW * CB


def _scale_chunk(rows, ewv_all, off):
    def grp(g, _):
        base = off + g * 16
        for e in range(16):
            wb = plsc.load_gather(ewv_all, [jnp.zeros((16,), _i32)
                                            + (base + e)])
            i = g * 16 + e
            for k in range(D // 16):
                rows[i, pl.ds(16 * k, 16)] = (
                    rows[i, pl.ds(16 * k, 16)] * wb)
        return 0

    lax.fori_loop(0, CB // 16, grp, 0)


def _gcn_body(hx_hbm, src_hbm, dst_hbm, ew_hbm,
              out_hbm,
              acc_sh, zbuf, rows0, rows1, ewv_all,
              sbuf0, sbuf1, dbuf0, dbuf1,
              sem0, sem1, sem2, sem3, sem4, sem5, sem6, sem7):
    c = lax.axis_index("c")
    s = lax.axis_index("s")
    w = c * NS + s
    ebase = w * EPB

    dew = pltpu.async_copy(ew_hbm.at[pl.ds(ebase, EPB)], ewv_all, sem2)

    def zrow(i, _):
        for k in range(D // 16):
            zbuf[i, pl.ds(16 * k, 16)] = jnp.zeros((16,), _f32)
        return 0

    lax.fori_loop(0, 16, zrow, 0)
    zds = [pltpu.async_copy(zbuf, acc_sh.at[pl.ds(s * 624 + j * 16, 16)],
                            sem0)
           for j in range(39)]
    for zd in zds:
        zd.wait()

    @pl.when(s == 0)
    def _():
        pltpu.sync_copy(zbuf, acc_sh.at[pl.ds(9984, 16)])

    dew.wait()
    plsc.subcore_barrier()

    def body(i, _):
        oa = (2 * i) * CB
        ob = (2 * i + 1) * CB
        ss0 = pltpu.async_copy(
            src_hbm.at[pl.ds(ebase + oa, CB)], sbuf0, sem6)
        ss1 = pltpu.async_copy(
            src_hbm.at[pl.ds(ebase + ob, CB)], sbuf1, sem7)
        dd0 = pltpu.async_copy(
            dst_hbm.at[pl.ds(ebase + oa, CB)], dbuf0, sem4)
        dd1 = pltpu.async_copy(
            dst_hbm.at[pl.ds(ebase + ob, CB)], dbuf1, sem5)
        ss0.wait()
        ga = pltpu.async_copy(hx_hbm.at[sbuf0], rows0, sem0)
        ss1.wait()
        gb = pltpu.async_copy(hx_hbm.at[sbuf1], rows1, sem1)
        ga.wait()
        _scale_chunk(rows0, ewv_all, oa)
        dd0.wait()
        sa = pltpu.async_copy(rows0, acc_sh.at[dbuf0], sem2, add=True)
        gb.wait()
        _scale_chunk(rows1, ewv_all, ob)
        dd1.wait()
        sb = pltpu.async_copy(rows1, acc_sh.at[dbuf1], sem3, add=True)
        sa.wait()
        sb.wait()
        return 0

    lax.fori_loop(0, CPW // 2, body, 0)

    @pl.when(w < 4)
    def _():
        bt = NW * EPB + w * CB
        pltpu.sync_copy(src_hbm.at[pl.ds(bt, CB)], sbuf0)
        pltpu.sync_copy(dst_hbm.at[pl.ds(bt, CB)], dbuf0)
        pltpu.sync_copy(ew_hbm.at[pl.ds(bt, CB)],
                        ewv_all.at[pl.ds(0, CB)])
        pltpu.sync_copy(hx_hbm.at[sbuf0], rows0)
        _scale_chunk(rows0, ewv_all, 0)
        pltpu.sync_copy(rows0, acc_sh.at[dbuf0], add=True)

    plsc.subcore_barrier()
    for j in range(6):
        o = s * 624 + j * RC
        pltpu.sync_copy(acc_sh.at[pl.ds(o, RC)],
                        out_hbm.at[c, pl.ds(o, RC)])

    @pl.when(s == 0)
    def _():
        pltpu.sync_copy(acc_sh.at[pl.ds(9984, 16)],
                        out_hbm.at[c, pl.ds(9984, 16)])


_gcn = pl.kernel(
    _gcn_body,
    out_type=jax.ShapeDtypeStruct((NC, N, D), _f32),
    mesh=_mesh,
    compiler_params=pltpu.CompilerParams(needs_layout_passes=False),
    scratch_types=[
        pltpu.VMEM_SHARED((N, D), _f32),
        pltpu.VMEM((16, D), _f32),
        pltpu.VMEM((CB, D), _f32),
        pltpu.VMEM((CB, D), _f32),
        pltpu.VMEM((EPB,), _f32),
        pltpu.VMEM((CB,), _i32),
        pltpu.VMEM((CB,), _i32),
        pltpu.VMEM((CB,), _i32),
        pltpu.VMEM((CB,), _i32),
        pltpu.SemaphoreType.DMA,
        pltpu.SemaphoreType.DMA,
        pltpu.SemaphoreType.DMA,
        pltpu.SemaphoreType.DMA,
        pltpu.SemaphoreType.DMA,
        pltpu.SemaphoreType.DMA,
        pltpu.SemaphoreType.DMA,
        pltpu.SemaphoreType.DMA,
    ],
)


def _pool_body(h_hbm, batch_hbm, out_hbm,
               acc_sh, zbuf, vals, bidx, vals16, bidx16):
    c = lax.axis_index("c")
    s = lax.axis_index("s")
    w = c * NS + s

    def zrow(i, _):
        for k in range(D // 16):
            zbuf[i, pl.ds(16 * k, 16)] = jnp.zeros((16,), _f32)
        return 0

    lax.fori_loop(0, 24, zrow, 0)
    pltpu.sync_copy(zbuf, acc_sh.at[pl.ds(s * 24, 24)])

    @pl.when(s == 0)
    def _():
        pltpu.sync_copy(zbuf.at[pl.ds(0, 16)], acc_sh.at[pl.ds(384, 16)])

    plsc.subcore_barrier()

    rbase = w * RPW

    def chunk(j, _):
        b = rbase + j * RC
        pltpu.sync_copy(h_hbm.at[pl.ds(b, RC)], vals)
        pltpu.sync_copy(batch_hbm.at[pl.ds(b, RC)], bidx)
        pltpu.sync_copy(vals, acc_sh.at[bidx], add=True)
        return 0

    lax.fori_loop(0, RPW // RC, chunk, 0)

    @pl.when(w == 0)
    def _():
        b = NW * RPW
        pltpu.sync_copy(h_hbm.at[pl.ds(b, 16)], vals16)
        pltpu.sync_copy(batch_hbm.at[pl.ds(b, 16)], bidx16)
        pltpu.sync_copy(vals16, acc_sh.at[bidx16], add=True)

    plsc.subcore_barrier()
    pltpu.sync_copy(acc_sh.at[pl.ds(s * 24, 24)],
                    out_hbm.at[c, pl.ds(s * 24, 24)])

    @pl.when(s == 0)
    def _():
        pltpu.sync_copy(acc_sh.at[pl.ds(384, 16)],
                        out_hbm.at[c, pl.ds(384, 16)])


_pool = pl.kernel(
    _pool_body,
    out_type=jax.ShapeDtypeStruct((NC, G, D), _f32),
    mesh=_mesh,
    compiler_params=pltpu.CompilerParams(needs_layout_passes=False),
    scratch_types=[
        pltpu.VMEM_SHARED((G, D), _f32),
        pltpu.VMEM((24, D), _f32),
        pltpu.VMEM((RC, D), _f32),
        pltpu.VMEM((RC,), _i32),
        pltpu.VMEM((16, D), _f32),
        pltpu.VMEM((16,), _i32),
    ],
)


BN = 2000
GRID = N // BN
_dims = (((1,), (1,)), ((), ()))


def _t0_body(h_ref, w_ref, b_ref, o_ref):
    hx = lax.dot_general(h_ref[...], w_ref[...], _dims,
                         preferred_element_type=_f32)
    o_ref[...] = jnp.maximum(hx + b_ref[...], 0.0)


_t0 = pl.pallas_call(
    _t0_body,
    grid=(GRID,),
    in_specs=[
        pl.BlockSpec((BN, D), lambda i: (i, 0)),
        pl.BlockSpec((D, D), lambda i: (0, 0)),
        pl.BlockSpec((1, D), lambda i: (0, 0)),
    ],
    out_specs=pl.BlockSpec((BN, D), lambda i: (i, 0)),
    out_shape=jax.ShapeDtypeStruct((N, D), _f32),
)


def _l2norm_rows(t):
    n = jnp.sqrt(jnp.sum(t * t, axis=1, keepdims=True))
    return t / jnp.maximum(n, 1e-12)


def _mid_body(acc_ref, h_ref, w_ref, b_ref, hn_ref, hx_ref):
    t = acc_ref[0] + acc_ref[1] + h_ref[...]
    h = _l2norm_rows(t)
    hn_ref[...] = h
    hx = lax.dot_general(h, w_ref[...], _dims, preferred_element_type=_f32)
    hx_ref[...] = jnp.maximum(hx + b_ref[...], 0.0)


_tmid = pl.pallas_call(
    _mid_body,
    grid=(GRID,),
    in_specs=[
        pl.BlockSpec((NC, BN, D), lambda i: (0, i, 0)),
        pl.BlockSpec((BN, D), lambda i: (i, 0)),
        pl.BlockSpec((D, D), lambda i: (0, 0)),
        pl.BlockSpec((1, D), lambda i: (0, 0)),
    ],
    out_specs=[
        pl.BlockSpec((BN, D), lambda i: (i, 0)),
        pl.BlockSpec((BN, D), lambda i: (i, 0)),
    ],
    out_shape=[
        jax.ShapeDtypeStruct((N, D), _f32),
        jax.ShapeDtypeStruct((N, D), _f32),
    ],
)


def _fin_body(acc_ref, h_ref, l0_ref, b0_ref, l1_ref, b1_ref, o_ref):
    t = acc_ref[0] + acc_ref[1] + h_ref[...]
    h = _l2norm_rows(t)
    a = lax.dot_general(h, l0_ref[...], _dims, preferred_element_type=_f32)
    a = jnp.maximum(a + b0_ref[...], 0.0)
    a = lax.dot_general(a, l1_ref[...], _dims, preferred_element_type=_f32)
    o_ref[...] = jnp.maximum(a + b1_ref[...], 0.0)


_tfin = pl.pallas_call(
    _fin_body,
    grid=(GRID,),
    in_specs=[
        pl.BlockSpec((NC, BN, D), lambda i: (0, i, 0)),
        pl.BlockSpec((BN, D), lambda i: (i, 0)),
        pl.BlockSpec((D, D), lambda i: (0, 0)),
        pl.BlockSpec((1, D), lambda i: (0, 0)),
        pl.BlockSpec((D, D), lambda i: (0, 0)),
        pl.BlockSpec((1, D), lambda i: (0, 0)),
    ],
    out_specs=pl.BlockSpec((BN, D), lambda i: (i, 0)),
    out_shape=jax.ShapeDtypeStruct((N, D), _f32),
)


def _head_body(p_ref, wp_ref, bp_ref, o_ref):
    m = p_ref[0] + p_ref[1]
    o = jnp.sum(m * wp_ref[...], axis=1, keepdims=True)
    o_ref[...] = o + bp_ref[0, 0]


_thead = pl.pallas_call(
    _head_body,
    out_shape=jax.ShapeDtypeStruct((G, 1), _f32),
)


def kernel(x, edge_index, edge_attr, batch, embd, gammas, waW, waB,
           linW, linB, wpW, wpB):
    src = edge_index[0]
    dst = edge_index[1]
    ea = edge_attr[:, 0]
    gp = jnp.pad(gammas[:, :, 0], ((0, 0), (0, VP - V))).reshape(-1)

    h0, ew0, ew1, ew2 = _prep(x, ea, dst, gp, embd)
    ews = (ew0, ew1, ew2)
    h = h0
    hx = _t0(h0, waW[0], waB[0].reshape(1, D))
    for l in range(3):
        acc = _gcn(hx, src, dst, ews[l])
        if l < 2:
            h, hx = _tmid(acc, h, waW[l + 1], waB[l + 1].reshape(1, D))
        else:
            hfin = _tfin(acc, h, linW[0], linB[0].reshape(1, D),
                         linW[1], linB[1].reshape(1, D))
    part = _pool(hfin, batch)
    props = _thead(part, wpW, wpB.reshape(1, 1))
    return props[:, 0]


# speedup vs baseline: 1.0669x; 1.0669x over previous
import functools

import jax
import jax.numpy as jnp
from jax import lax
from jax.experimental import pallas as pl
from jax.experimental.pallas import tpu as pltpu
from jax.experimental.pallas import tpu_sc as plsc

N = 10000
E = 320000
D = 128
V = 100
VP = 112
G = 400
NC = 2
NS = 16
NW = NC * NS
RPW = 312
RC = 104
EPW = E // NW
EC = 80
NCH = EPW // EC

_mesh = plsc.VectorSubcoreMesh(core_axis_name="c", subcore_axis_name="s")
_f32 = jnp.float32
_i32 = jnp.int32


EPB0 = 9984


def _prep_body(x_hbm, ea_hbm, dst_hbm, gp_hbm, embd_hbm,
               h0_hbm, ew0_hbm, ew1_hbm, ew2_hbm,
               embd_sh, gpv, gtab, xv0, xv1, xv2, hrows0, hrows1, hrows2,
               xv16, hrows16,
               didx_all, av_all, xdv_all, ewb0, ewb1, ewb2,
               sem0, sem1, sem2, sem3):
    c = lax.axis_index("c")
    s = lax.axis_index("s")
    w = c * NS + s
    ew_hbms = (ew0_hbm, ew1_hbm, ew2_hbm)
    ewbs = (ewb0, ewb1, ewb2)
    ebase = w * EPB0

    dd = pltpu.async_copy(dst_hbm.at[pl.ds(ebase, EPB0)], didx_all, sem0)
    de = pltpu.async_copy(ea_hbm.at[pl.ds(ebase, EPB0)], av_all, sem1)

    @pl.when(s == 0)
    def _():
        pltpu.sync_copy(embd_hbm, embd_sh)

    pltpu.sync_copy(gp_hbm, gpv)
    for i in range(3 * VP // 16):
        v = gpv[pl.ds(16 * i, 16)]
        gtab[pl.ds(16 * i, 16)] = 1.0 / (1.0 + jnp.exp(-v))
    plsc.subcore_barrier()

    rbase = w * RPW
    hrows = (hrows0, hrows1, hrows2)
    xvs = (xv0, xv1, xv2)
    xls = [pltpu.async_copy(x_hbm.at[pl.ds(rbase + j * RC, RC)], xvs[j],
                            sem2)
           for j in range(3)]
    for xl in xls:
        xl.wait()
    gds = [pltpu.async_copy(embd_sh.at[xvs[j]], hrows[j], sem2)
           for j in range(3)]
    for g in gds:
        g.wait()
    hds = [pltpu.async_copy(hrows[j], h0_hbm.at[pl.ds(rbase + j * RC, RC)],
                            sem2)
           for j in range(3)]
    for h in hds:
        h.wait()

    @pl.when(w == 0)
    def _():
        b = NW * RPW
        pltpu.sync_copy(x_hbm.at[pl.ds(b, 16)], xv16)
        pltpu.sync_copy(embd_sh.at[xv16], hrows16)
        pltpu.sync_copy(hrows16, h0_hbm.at[pl.ds(b, 16)])

    dd.wait()
    xds = [pltpu.async_copy(
        x_hbm.at[didx_all.at[pl.ds(j * 128, 128)]],
        xdv_all.at[pl.ds(j * 128, 128)], sem3)
        for j in range(EPB0 // 128)]
    for xd in xds:
        xd.wait()
    de.wait()

    def grp(g, _):
        a = av_all[pl.ds(g * 16, 16)]
        xi = xdv_all[pl.ds(g * 16, 16)]
        a2 = a * a
        for l in range(3):
            gl = plsc.load_gather(gtab, [xi + (l * VP)])
            ewbs[l][pl.ds(g * 16, 16)] = jnp.exp(-gl * a2)
        return 0

    lax.fori_loop(0, EPB0 // 16, grp, 0)
    eds = [pltpu.async_copy(ewbs[l], ew_hbms[l].at[pl.ds(ebase, EPB0)],
                            sem2)
           for l in range(3)]
    for e in eds:
        e.wait()

    @pl.when(w < 4)
    def _():
        bt = NW * EPB0 + w * 128
        pltpu.sync_copy(dst_hbm.at[pl.ds(bt, 128)],
                        didx_all.at[pl.ds(0, 128)])
        pltpu.sync_copy(ea_hbm.at[pl.ds(bt, 128)],
                        av_all.at[pl.ds(0, 128)])
        pltpu.sync_copy(x_hbm.at[didx_all.at[pl.ds(0, 128)]],
                        xdv_all.at[pl.ds(0, 128)])
        lax.fori_loop(0, 8, grp, 0)
        for l in range(3):
            pltpu.sync_copy(ewbs[l].at[pl.ds(0, 128)],
                            ew_hbms[l].at[pl.ds(bt, 128)])


_prep = pl.kernel(
    _prep_body,
    out_type=(jax.ShapeDtypeStruct((N, D), _f32),
              jax.ShapeDtypeStruct((E,), _f32),
              jax.ShapeDtypeStruct((E,), _f32),
              jax.ShapeDtypeStruct((E,), _f32)),
    mesh=_mesh,
    compiler_params=pltpu.CompilerParams(needs_layout_passes=False),
    scratch_types=[
        pltpu.VMEM_SHARED((V, D), _f32),
        pltpu.VMEM((3 * VP,), _f32),
        pltpu.VMEM((3 * VP,), _f32),
        pltpu.VMEM((RC,), _i32),
        pltpu.VMEM((RC,), _i32),
        pltpu.VMEM((RC,), _i32),
        pltpu.VMEM((RC, D), _f32),
        pltpu.VMEM((RC, D), _f32),
        pltpu.VMEM((RC, D), _f32),
        pltpu.VMEM((16,), _i32),
        pltpu.VMEM((16, D), _f32),
        pltpu.VMEM((EPB0,), _i32),
        pltpu.VMEM((EPB0,), _f32),
        pltpu.VMEM((EPB0,), _i32),
        pltpu.VMEM((EPB0,), _f32),
        pltpu.VMEM((EPB0,), _f32),
        pltpu.VMEM((EPB0,), _f32),
        pltpu.SemaphoreType.DMA,
        pltpu.SemaphoreType.DMA,
        pltpu.SemaphoreType.DMA,
        pltpu.SemaphoreType.DMA,
    ],
)


CB = 128
CPW = 78
EPB = CPW * CB


def _scale_chunk(rows, ewv_all, off):
    def grp(g, _):
        base = off + g * 16
        for e in range(16):
            wb = plsc.load_gather(ewv_all, [jnp.zeros((16,), _i32)
                                            + (base + e)])
            i = g * 16 + e
            for k in range(D // 16):
                rows[i, pl.ds(16 * k, 16)] = (
                    rows[i, pl.ds(16 * k, 16)] * wb)
        return 0

    lax.fori_loop(0, CB // 16, grp, 0)


def _gcn_body(hx_hbm, src_hbm, dst_hbm, ew_hbm,
              out_hbm,
              acc_sh, zbuf, rows0, rows1, ewv_all,
              sbuf0, sbuf1, dbuf0, dbuf1,
              sem0, sem1, sem2, sem3, sem4, sem5, sem6, sem7):
    c = lax.axis_index("c")
    s = lax.axis_index("s")
    w = c * NS + s
    ebase = w * EPB

    dew = pltpu.async_copy(ew_hbm.at[pl.ds(ebase, EPB)], ewv_all, sem2)

    def zrow(i, _):
        for k in range(D // 16):
            zbuf[i, pl.ds(16 * k, 16)] = jnp.zeros((16,), _f32)
        return 0

    lax.fori_loop(0, 16, zrow, 0)
    zds = [pltpu.async_copy(zbuf, acc_sh.at[pl.ds(s * 624 + j * 16, 16)],
                            sem0)
           for j in range(39)]
    for zd in zds:
        zd.wait()

    @pl.when(s == 0)
    def _():
        pltpu.sync_copy(zbuf, acc_sh.at[pl.ds(9984, 16)])

    dew.wait()
    plsc.subcore_barrier()

    pltpu.async_copy(src_hbm.at[pl.ds(ebase, CB)], sbuf0, sem6)
    pltpu.async_copy(src_hbm.at[pl.ds(ebase + CB, CB)], sbuf1, sem7)

    def body(i, _):
        oa = (2 * i) * CB
        ob = (2 * i + 1) * CB

        @pl.when(i > 0)
        def _():
            pltpu.make_async_copy(rows0, acc_sh.at[dbuf0], sem2).wait()
            pltpu.make_async_copy(rows1, acc_sh.at[dbuf1], sem3).wait()

        dd0 = pltpu.async_copy(
            dst_hbm.at[pl.ds(ebase + oa, CB)], dbuf0, sem4)
        dd1 = pltpu.async_copy(
            dst_hbm.at[pl.ds(ebase + ob, CB)], dbuf1, sem5)
        pltpu.make_async_copy(
            src_hbm.at[pl.ds(ebase + oa, CB)], sbuf0, sem6).wait()
        ga = pltpu.async_copy(hx_hbm.at[sbuf0], rows0, sem0)
        pltpu.make_async_copy(
            src_hbm.at[pl.ds(ebase + ob, CB)], sbuf1, sem7).wait()
        gb = pltpu.async_copy(hx_hbm.at[sbuf1], rows1, sem1)
        ga.wait()
        pltpu.async_copy(
            src_hbm.at[pl.ds(ebase + oa + 2 * CB, CB)], sbuf0, sem6)
        _scale_chunk(rows0, ewv_all, oa)
        dd0.wait()
        pltpu.async_copy(rows0, acc_sh.at[dbuf0], sem2, add=True)
        gb.wait()
        pltpu.async_copy(
            src_hbm.at[pl.ds(ebase + ob + 2 * CB, CB)], sbuf1, sem7)
        _scale_chunk(rows1, ewv_all, ob)
        dd1.wait()
        pltpu.async_copy(rows1, acc_sh.at[dbuf1], sem3, add=True)
        return 0

    lax.fori_loop(0, CPW // 2, body, 0)
    pltpu.make_async_copy(rows0, acc_sh.at[dbuf0], sem2).wait()
    pltpu.make_async_copy(rows1, acc_sh.at[dbuf1], sem3).wait()
    pltpu.make_async_copy(src_hbm.at[pl.ds(ebase, CB)], sbuf0, sem6).wait()
    pltpu.make_async_copy(src_hbm.at[pl.ds(ebase, CB)], sbuf1, sem7).wait()

    @pl.when(w < 4)
    def _():
        bt = NW * EPB + w * CB
        pltpu.sync_copy(src_hbm.at[pl.ds(bt, CB)], sbuf0)
        pltpu.sync_copy(dst_hbm.at[pl.ds(bt, CB)], dbuf0)
        pltpu.sync_copy(ew_hbm.at[pl.ds(bt, CB)],
                        ewv_all.at[pl.ds(0, CB)])
        pltpu.sync_copy(hx_hbm.at[sbuf0], rows0)
        _scale_chunk(rows0, ewv_all, 0)
        pltpu.sync_copy(rows0, acc_sh.at[dbuf0], add=True)

    plsc.subcore_barrier()
    for j in range(6):
        o = s * 624 + j * RC
        pltpu.sync_copy(acc_sh.at[pl.ds(o, RC)],
                        out_hbm.at[c, pl.ds(o, RC)])

    @pl.when(s == 0)
    def _():
        pltpu.sync_copy(acc_sh.at[pl.ds(9984, 16)],
                        out_hbm.at[c, pl.ds(9984, 16)])


_gcn = pl.kernel(
    _gcn_body,
    out_type=jax.ShapeDtypeStruct((NC, N, D), _f32),
    mesh=_mesh,
    compiler_params=pltpu.CompilerParams(needs_layout_passes=False),
    scratch_types=[
        pltpu.VMEM_SHARED((N, D), _f32),
        pltpu.VMEM((16, D), _f32),
        pltpu.VMEM((CB, D), _f32),
        pltpu.VMEM((CB, D), _f32),
        pltpu.VMEM((EPB,), _f32),
        pltpu.VMEM((CB,), _i32),
        pltpu.VMEM((CB,), _i32),
        pltpu.VMEM((CB,), _i32),
        pltpu.VMEM((CB,), _i32),
        pltpu.SemaphoreType.DMA,
        pltpu.SemaphoreType.DMA,
        pltpu.SemaphoreType.DMA,
        pltpu.SemaphoreType.DMA,
        pltpu.SemaphoreType.DMA,
        pltpu.SemaphoreType.DMA,
        pltpu.SemaphoreType.DMA,
        pltpu.SemaphoreType.DMA,
    ],
)


def _pool_body(h_hbm, batch_hbm, out_hbm,
               acc_sh, zbuf, vals, bidx, vals16, bidx16):
    c = lax.axis_index("c")
    s = lax.axis_index("s")
    w = c * NS + s

    def zrow(i, _):
        for k in range(D // 16):
            zbuf[i, pl.ds(16 * k, 16)] = jnp.zeros((16,), _f32)
        return 0

    lax.fori_loop(0, 24, zrow, 0)
    pltpu.sync_copy(zbuf, acc_sh.at[pl.ds(s * 24, 24)])

    @pl.when(s == 0)
    def _():
        pltpu.sync_copy(zbuf.at[pl.ds(0, 16)], acc_sh.at[pl.ds(384, 16)])

    plsc.subcore_barrier()

    rbase = w * RPW

    def chunk(j, _):
        b = rbase + j * RC
        pltpu.sync_copy(h_hbm.at[pl.ds(b, RC)], vals)
        pltpu.sync_copy(batch_hbm.at[pl.ds(b, RC)], bidx)
        pltpu.sync_copy(vals, acc_sh.at[bidx], add=True)
        return 0

    lax.fori_loop(0, RPW // RC, chunk, 0)

    @pl.when(w == 0)
    def _():
        b = NW * RPW
        pltpu.sync_copy(h_hbm.at[pl.ds(b, 16)], vals16)
        pltpu.sync_copy(batch_hbm.at[pl.ds(b, 16)], bidx16)
        pltpu.sync_copy(vals16, acc_sh.at[bidx16], add=True)

    plsc.subcore_barrier()
    pltpu.sync_copy(acc_sh.at[pl.ds(s * 24, 24)],
                    out_hbm.at[c, pl.ds(s * 24, 24)])

    @pl.when(s == 0)
    def _():
        pltpu.sync_copy(acc_sh.at[pl.ds(384, 16)],
                        out_hbm.at[c, pl.ds(384, 16)])


_pool = pl.kernel(
    _pool_body,
    out_type=jax.ShapeDtypeStruct((NC, G, D), _f32),
    mesh=_mesh,
    compiler_params=pltpu.CompilerParams(needs_layout_passes=False),
    scratch_types=[
        pltpu.VMEM_SHARED((G, D), _f32),
        pltpu.VMEM((24, D), _f32),
        pltpu.VMEM((RC, D), _f32),
        pltpu.VMEM((RC,), _i32),
        pltpu.VMEM((16, D), _f32),
        pltpu.VMEM((16,), _i32),
    ],
)


BN = 2000
GRID = N // BN
_dims = (((1,), (1,)), ((), ()))


def _t0_body(h_ref, w_ref, b_ref, o_ref):
    hx = lax.dot_general(h_ref[...], w_ref[...], _dims,
                         preferred_element_type=_f32)
    o_ref[...] = jnp.maximum(hx + b_ref[...], 0.0)


_t0 = pl.pallas_call(
    _t0_body,
    grid=(GRID,),
    in_specs=[
        pl.BlockSpec((BN, D), lambda i: (i, 0)),
        pl.BlockSpec((D, D), lambda i: (0, 0)),
        pl.BlockSpec((1, D), lambda i: (0, 0)),
    ],
    out_specs=pl.BlockSpec((BN, D), lambda i: (i, 0)),
    out_shape=jax.ShapeDtypeStruct((N, D), _f32),
)


def _l2norm_rows(t):
    n = jnp.sqrt(jnp.sum(t * t, axis=1, keepdims=True))
    return t / jnp.maximum(n, 1e-12)


def _mid_body(acc_ref, h_ref, w_ref, b_ref, hn_ref, hx_ref):
    t = acc_ref[0] + acc_ref[1] + h_ref[...]
    h = _l2norm_rows(t)
    hn_ref[...] = h
    hx = lax.dot_general(h, w_ref[...], _dims, preferred_element_type=_f32)
    hx_ref[...] = jnp.maximum(hx + b_ref[...], 0.0)


_tmid = pl.pallas_call(
    _mid_body,
    grid=(GRID,),
    in_specs=[
        pl.BlockSpec((NC, BN, D), lambda i: (0, i, 0)),
        pl.BlockSpec((BN, D), lambda i: (i, 0)),
        pl.BlockSpec((D, D), lambda i: (0, 0)),
        pl.BlockSpec((1, D), lambda i: (0, 0)),
    ],
    out_specs=[
        pl.BlockSpec((BN, D), lambda i: (i, 0)),
        pl.BlockSpec((BN, D), lambda i: (i, 0)),
    ],
    out_shape=[
        jax.ShapeDtypeStruct((N, D), _f32),
        jax.ShapeDtypeStruct((N, D), _f32),
    ],
)


def _fin_body(acc_ref, h_ref, l0_ref, b0_ref, l1_ref, b1_ref, o_ref):
    t = acc_ref[0] + acc_ref[1] + h_ref[...]
    h = _l2norm_rows(t)
    a = lax.dot_general(h, l0_ref[...], _dims, preferred_element_type=_f32)
    a = jnp.maximum(a + b0_ref[...], 0.0)
    a = lax.dot_general(a, l1_ref[...], _dims, preferred_element_type=_f32)
    o_ref[...] = jnp.maximum(a + b1_ref[...], 0.0)


_tfin = pl.pallas_call(
    _fin_body,
    grid=(GRID,),
    in_specs=[
        pl.BlockSpec((NC, BN, D), lambda i: (0, i, 0)),
        pl.BlockSpec((BN, D), lambda i: (i, 0)),
        pl.BlockSpec((D, D), lambda i: (0, 0)),
        pl.BlockSpec((1, D), lambda i: (0, 0)),
        pl.BlockSpec((D, D), lambda i: (0, 0)),
        pl.BlockSpec((1, D), lambda i: (0, 0)),
    ],
    out_specs=pl.BlockSpec((BN, D), lambda i: (i, 0)),
    out_shape=jax.ShapeDtypeStruct((N, D), _f32),
)


def _head_body(p_ref, wp_ref, bp_ref, o_ref):
    m = p_ref[0] + p_ref[1]
    o = jnp.sum(m * wp_ref[...], axis=1, keepdims=True)
    o_ref[...] = o + bp_ref[0, 0]


_thead = pl.pallas_call(
    _head_body,
    out_shape=jax.ShapeDtypeStruct((G, 1), _f32),
)


def kernel(x, edge_index, edge_attr, batch, embd, gammas, waW, waB,
           linW, linB, wpW, wpB):
    src = edge_index[0]
    dst = edge_index[1]
    ea = edge_attr[:, 0]
    gp = jnp.pad(gammas[:, :, 0], ((0, 0), (0, VP - V))).reshape(-1)

    h0, ew0, ew1, ew2 = _prep(x, ea, dst, gp, embd)
    ews = (ew0, ew1, ew2)
    h = h0
    hx = _t0(h0, waW[0], waB[0].reshape(1, D))
    for l in range(3):
        acc = _gcn(hx, src, dst, ews[l])
        if l < 2:
            h, hx = _tmid(acc, h, waW[l + 1], waB[l + 1].reshape(1, D))
        else:
            hfin = _tfin(acc, h, linW[0], linB[0].reshape(1, D),
                         linW[1], linB[1].reshape(1, D))
    part = _pool(hfin, batch)
    props = _thead(part, wpW, wpB.reshape(1, 1))
    return props[:, 0]


# speedup vs baseline: 1.2079x; 1.1322x over previous
import functools

import jax
import jax.numpy as jnp
from jax import lax
from jax.experimental import pallas as pl
from jax.experimental.pallas import tpu as pltpu
from jax.experimental.pallas import tpu_sc as plsc

N = 10000
E = 320000
D = 128
V = 100
VP = 112
G = 400
NC = 2
NS = 16
NW = NC * NS
RPW = 312
RC = 104
EPW = E // NW
EC = 80
NCH = EPW // EC

_mesh = plsc.VectorSubcoreMesh(core_axis_name="c", subcore_axis_name="s")
_f32 = jnp.float32
_i32 = jnp.int32


EPB0 = 9984


def _prep_body(x_hbm, ea_hbm, dst_hbm, gp_hbm, embd_hbm,
               h0_hbm, ew0_hbm, ew1_hbm, ew2_hbm,
               embd_sh, gpv, gtab, xv0, xv1, xv2, hrows0, hrows1, hrows2,
               xv16, hrows16,
               didx_all, av_all, xdv_all, ewb0, ewb1, ewb2,
               sem0, sem1, sem2, sem3):
    c = lax.axis_index("c")
    s = lax.axis_index("s")
    w = c * NS + s
    ew_hbms = (ew0_hbm, ew1_hbm, ew2_hbm)
    ewbs = (ewb0, ewb1, ewb2)
    ebase = w * EPB0

    dd = pltpu.async_copy(dst_hbm.at[pl.ds(ebase, EPB0)], didx_all, sem0)
    de = pltpu.async_copy(ea_hbm.at[pl.ds(ebase, EPB0)], av_all, sem1)

    @pl.when(s == 0)
    def _():
        pltpu.sync_copy(embd_hbm, embd_sh)

    pltpu.sync_copy(gp_hbm, gpv)
    for i in range(3 * VP // 16):
        v = gpv[pl.ds(16 * i, 16)]
        gtab[pl.ds(16 * i, 16)] = 1.0 / (1.0 + jnp.exp(-v))
    plsc.subcore_barrier()

    rbase = w * RPW
    hrows = (hrows0, hrows1, hrows2)
    xvs = (xv0, xv1, xv2)
    xls = [pltpu.async_copy(x_hbm.at[pl.ds(rbase + j * RC, RC)], xvs[j],
                            sem2)
           for j in range(3)]
    for xl in xls:
        xl.wait()
    gds = [pltpu.async_copy(embd_sh.at[xvs[j]], hrows[j], sem2)
           for j in range(3)]
    for g in gds:
        g.wait()
    hds = [pltpu.async_copy(hrows[j], h0_hbm.at[pl.ds(rbase + j * RC, RC)],
                            sem2)
           for j in range(3)]
    for h in hds:
        h.wait()

    @pl.when(w == 0)
    def _():
        b = NW * RPW
        pltpu.sync_copy(x_hbm.at[pl.ds(b, 16)], xv16)
        pltpu.sync_copy(embd_sh.at[xv16], hrows16)
        pltpu.sync_copy(hrows16, h0_hbm.at[pl.ds(b, 16)])

    dd.wait()
    xds = [pltpu.async_copy(
        x_hbm.at[didx_all.at[pl.ds(j * 128, 128)]],
        xdv_all.at[pl.ds(j * 128, 128)], sem3)
        for j in range(EPB0 // 128)]
    for xd in xds:
        xd.wait()
    de.wait()

    def grp(g, _):
        a = av_all[pl.ds(g * 16, 16)]
        xi = xdv_all[pl.ds(g * 16, 16)]
        a2 = a * a
        for l in range(3):
            gl = plsc.load_gather(gtab, [xi + (l * VP)])
            ewbs[l][pl.ds(g * 16, 16)] = jnp.exp(-gl * a2)
        return 0

    lax.fori_loop(0, EPB0 // 16, grp, 0)
    eds = [pltpu.async_copy(ewbs[l], ew_hbms[l].at[pl.ds(ebase, EPB0)],
                            sem2)
           for l in range(3)]
    for e in eds:
        e.wait()

    @pl.when(w < 4)
    def _():
        bt = NW * EPB0 + w * 128
        pltpu.sync_copy(dst_hbm.at[pl.ds(bt, 128)],
                        didx_all.at[pl.ds(0, 128)])
        pltpu.sync_copy(ea_hbm.at[pl.ds(bt, 128)],
                        av_all.at[pl.ds(0, 128)])
        pltpu.sync_copy(x_hbm.at[didx_all.at[pl.ds(0, 128)]],
                        xdv_all.at[pl.ds(0, 128)])
        lax.fori_loop(0, 8, grp, 0)
        for l in range(3):
            pltpu.sync_copy(ewbs[l].at[pl.ds(0, 128)],
                            ew_hbms[l].at[pl.ds(bt, 128)])


_prep = pl.kernel(
    _prep_body,
    out_type=(jax.ShapeDtypeStruct((N, D), _f32),
              jax.ShapeDtypeStruct((E,), _f32),
              jax.ShapeDtypeStruct((E,), _f32),
              jax.ShapeDtypeStruct((E,), _f32)),
    mesh=_mesh,
    compiler_params=pltpu.CompilerParams(needs_layout_passes=False),
    scratch_types=[
        pltpu.VMEM_SHARED((V, D), _f32),
        pltpu.VMEM((3 * VP,), _f32),
        pltpu.VMEM((3 * VP,), _f32),
        pltpu.VMEM((RC,), _i32),
        pltpu.VMEM((RC,), _i32),
        pltpu.VMEM((RC,), _i32),
        pltpu.VMEM((RC, D), _f32),
        pltpu.VMEM((RC, D), _f32),
        pltpu.VMEM((RC, D), _f32),
        pltpu.VMEM((16,), _i32),
        pltpu.VMEM((16, D), _f32),
        pltpu.VMEM((EPB0,), _i32),
        pltpu.VMEM((EPB0,), _f32),
        pltpu.VMEM((EPB0,), _i32),
        pltpu.VMEM((EPB0,), _f32),
        pltpu.VMEM((EPB0,), _f32),
        pltpu.VMEM((EPB0,), _f32),
        pltpu.SemaphoreType.DMA,
        pltpu.SemaphoreType.DMA,
        pltpu.SemaphoreType.DMA,
        pltpu.SemaphoreType.DMA,
    ],
)


CB = 104
CPW = 96
EPB = CPW * CB
TEPW = 64


def _scale_chunk(rows, ewc, nedge):
    def grp(g, _):
        for e in range(16):
            wb = plsc.load_gather(ewc, [jnp.zeros((16,), _i32)
                                        + (g * 16 + e)])
            i = g * 16 + e
            for k in range(D // 16):
                rows[i, pl.ds(16 * k, 16)] = (
                    rows[i, pl.ds(16 * k, 16)] * wb)
        return 0

    lax.fori_loop(0, nedge // 16, grp, 0)
    for e in range(nedge % 16):
        i = (nedge // 16) * 16 + e
        wb = plsc.load_gather(ewc, [jnp.zeros((16,), _i32) + i])
        for k in range(D // 16):
            rows[i, pl.ds(16 * k, 16)] = rows[i, pl.ds(16 * k, 16)] * wb


def _gcn_body(hx_hbm, src_hbm, dst_hbm, ew_hbm,
              out_hbm,
              acc_sh, zbuf, rows0, rows1, rows2,
              sbuf0, sbuf1, sbuf2, dbuf0, dbuf1, dbuf2,
              ewc0, ewc1, ewc2, tsb, tdb, tew,
              lsem0, lsem1, lsem2, gsem0, gsem1, gsem2,
              ssem0, ssem1, ssem2, dsem0, dsem1, dsem2):
    c = lax.axis_index("c")
    s = lax.axis_index("s")
    w = c * NS + s
    ebase = w * EPB
    rows = (rows0, rows1, rows2)
    sbuf = (sbuf0, sbuf1, sbuf2)
    dbuf = (dbuf0, dbuf1, dbuf2)
    ewc = (ewc0, ewc1, ewc2)
    lsem = (lsem0, lsem1, lsem2)
    gsem = (gsem0, gsem1, gsem2)
    ssem = (ssem0, ssem1, ssem2)
    dsem = (dsem0, dsem1, dsem2)

    def zrow(i, _):
        for k in range(D // 16):
            zbuf[i, pl.ds(16 * k, 16)] = jnp.zeros((16,), _f32)
        return 0

    lax.fori_loop(0, 16, zrow, 0)
    zds = [pltpu.async_copy(zbuf, acc_sh.at[pl.ds(s * 624 + j * 16, 16)],
                            gsem0)
           for j in range(39)]
    for zd in zds:
        zd.wait()

    @pl.when(s == 0)
    def _():
        pltpu.sync_copy(zbuf, acc_sh.at[pl.ds(9984, 16)])

    plsc.subcore_barrier()

    for t in range(3):
        pltpu.async_copy(src_hbm.at[pl.ds(ebase + t * CB, CB)], sbuf[t],
                         lsem[t])
        pltpu.async_copy(ew_hbm.at[pl.ds(ebase + t * CB, CB)], ewc[t],
                         lsem[t])

    def body(b, _):
        gds = []
        for t in range(3):
            off = (3 * b + t) * CB
            @pl.when(b > 0)
            def _():
                pltpu.make_async_copy(rows[t], acc_sh.at[dbuf[t]],
                                      ssem[t]).wait()

            dl = pltpu.async_copy(dst_hbm.at[pl.ds(ebase + off, CB)],
                                  dbuf[t], dsem[t])
            pltpu.make_async_copy(src_hbm.at[pl.ds(ebase + off, CB)],
                                  sbuf[t], lsem[t]).wait()
            pltpu.make_async_copy(ew_hbm.at[pl.ds(ebase + off, CB)],
                                  ewc[t], lsem[t]).wait()
            gds.append((pltpu.async_copy(hx_hbm.at[sbuf[t]], rows[t],
                                         gsem[t]), dl))
        for t in range(3):
            off = (3 * b + t) * CB
            g, dl = gds[t]
            g.wait()
            pltpu.async_copy(src_hbm.at[pl.ds(ebase + off + 3 * CB, CB)],
                             sbuf[t], lsem[t])
            _scale_chunk(rows[t], ewc[t], CB)
            pltpu.async_copy(ew_hbm.at[pl.ds(ebase + off + 3 * CB, CB)],
                             ewc[t], lsem[t])
            dl.wait()
            pltpu.async_copy(rows[t], acc_sh.at[dbuf[t]], ssem[t],
                             add=True)
        return 0

    lax.fori_loop(0, CPW // 3, body, 0)
    for t in range(3):
        pltpu.make_async_copy(rows[t], acc_sh.at[dbuf[t]], ssem[t]).wait()
        pltpu.make_async_copy(src_hbm.at[pl.ds(ebase, CB)], sbuf[t],
                              lsem[t]).wait()
        pltpu.make_async_copy(ew_hbm.at[pl.ds(ebase, CB)], ewc[t],
                              lsem[t]).wait()

    @pl.when(w < 8)
    def _():
        bt = NW * EPB + w * TEPW
        pltpu.sync_copy(src_hbm.at[pl.ds(bt, TEPW)], tsb)
        pltpu.sync_copy(dst_hbm.at[pl.ds(bt, TEPW)], tdb)
        pltpu.sync_copy(ew_hbm.at[pl.ds(bt, TEPW)], tew)
        pltpu.sync_copy(hx_hbm.at[tsb], rows0.at[pl.ds(0, TEPW)])
        _scale_chunk(rows0, tew, TEPW)
        pltpu.sync_copy(rows0.at[pl.ds(0, TEPW)], acc_sh.at[tdb],
                        add=True)

    plsc.subcore_barrier()
    for j in range(6):
        o = s * 624 + j * RC
        pltpu.sync_copy(acc_sh.at[pl.ds(o, RC)],
                        out_hbm.at[c, pl.ds(o, RC)])

    @pl.when(s == 0)
    def _():
        pltpu.sync_copy(acc_sh.at[pl.ds(9984, 16)],
                        out_hbm.at[c, pl.ds(9984, 16)])


_gcn = pl.kernel(
    _gcn_body,
    out_type=jax.ShapeDtypeStruct((NC, N, D), _f32),
    mesh=_mesh,
    compiler_params=pltpu.CompilerParams(needs_layout_passes=False),
    scratch_types=(
        [pltpu.VMEM_SHARED((N, D), _f32),
         pltpu.VMEM((16, D), _f32)]
        + [pltpu.VMEM((CB, D), _f32)] * 3
        + [pltpu.VMEM((CB,), _i32)] * 6
        + [pltpu.VMEM((CB,), _f32)] * 3
        + [pltpu.VMEM((TEPW,), _i32)] * 2
        + [pltpu.VMEM((TEPW,), _f32)]
        + [pltpu.SemaphoreType.DMA] * 12
    ),
)


def _pool_body(h_hbm, batch_hbm, out_hbm,
               acc_sh, zbuf, vals, bidx, vals16, bidx16):
    c = lax.axis_index("c")
    s = lax.axis_index("s")
    w = c * NS + s

    def zrow(i, _):
        for k in range(D // 16):
            zbuf[i, pl.ds(16 * k, 16)] = jnp.zeros((16,), _f32)
        return 0

    lax.fori_loop(0, 24, zrow, 0)
    pltpu.sync_copy(zbuf, acc_sh.at[pl.ds(s * 24, 24)])

    @pl.when(s == 0)
    def _():
        pltpu.sync_copy(zbuf.at[pl.ds(0, 16)], acc_sh.at[pl.ds(384, 16)])

    plsc.subcore_barrier()

    rbase = w * RPW

    def chunk(j, _):
        b = rbase + j * RC
        pltpu.sync_copy(h_hbm.at[pl.ds(b, RC)], vals)
        pltpu.sync_copy(batch_hbm.at[pl.ds(b, RC)], bidx)
        pltpu.sync_copy(vals, acc_sh.at[bidx], add=True)
        return 0

    lax.fori_loop(0, RPW // RC, chunk, 0)

    @pl.when(w == 0)
    def _():
        b = NW * RPW
        pltpu.sync_copy(h_hbm.at[pl.ds(b, 16)], vals16)
        pltpu.sync_copy(batch_hbm.at[pl.ds(b, 16)], bidx16)
        pltpu.sync_copy(vals16, acc_sh.at[bidx16], add=True)

    plsc.subcore_barrier()
    pltpu.sync_copy(acc_sh.at[pl.ds(s * 24, 24)],
                    out_hbm.at[c, pl.ds(s * 24, 24)])

    @pl.when(s == 0)
    def _():
        pltpu.sync_copy(acc_sh.at[pl.ds(384, 16)],
                        out_hbm.at[c, pl.ds(384, 16)])


_pool = pl.kernel(
    _pool_body,
    out_type=jax.ShapeDtypeStruct((NC, G, D), _f32),
    mesh=_mesh,
    compiler_params=pltpu.CompilerParams(needs_layout_passes=False),
    scratch_types=[
        pltpu.VMEM_SHARED((G, D), _f32),
        pltpu.VMEM((24, D), _f32),
        pltpu.VMEM((RC, D), _f32),
        pltpu.VMEM((RC,), _i32),
        pltpu.VMEM((16, D), _f32),
        pltpu.VMEM((16,), _i32),
    ],
)


BN = 2000
GRID = N // BN
_dims = (((1,), (1,)), ((), ()))


def _t0_body(h_ref, w_ref, b_ref, o_ref):
    hx = lax.dot_general(h_ref[...], w_ref[...], _dims,
                         preferred_element_type=_f32)
    o_ref[...] = jnp.maximum(hx + b_ref[...], 0.0)


_t0 = pl.pallas_call(
    _t0_body,
    grid=(GRID,),
    in_specs=[
        pl.BlockSpec((BN, D), lambda i: (i, 0)),
        pl.BlockSpec((D, D), lambda i: (0, 0)),
        pl.BlockSpec((1, D), lambda i: (0, 0)),
    ],
    out_specs=pl.BlockSpec((BN, D), lambda i: (i, 0)),
    out_shape=jax.ShapeDtypeStruct((N, D), _f32),
)


def _l2norm_rows(t):
    n = jnp.sqrt(jnp.sum(t * t, axis=1, keepdims=True))
    return t / jnp.maximum(n, 1e-12)


def _mid_body(acc_ref, h_ref, w_ref, b_ref, hn_ref, hx_ref):
    t = acc_ref[0] + acc_ref[1] + h_ref[...]
    h = _l2norm_rows(t)
    hn_ref[...] = h
    hx = lax.dot_general(h, w_ref[...], _dims, preferred_element_type=_f32)
    hx_ref[...] = jnp.maximum(hx + b_ref[...], 0.0)


_tmid = pl.pallas_call(
    _mid_body,
    grid=(GRID,),
    in_specs=[
        pl.BlockSpec((NC, BN, D), lambda i: (0, i, 0)),
        pl.BlockSpec((BN, D), lambda i: (i, 0)),
        pl.BlockSpec((D, D), lambda i: (0, 0)),
        pl.BlockSpec((1, D), lambda i: (0, 0)),
    ],
    out_specs=[
        pl.BlockSpec((BN, D), lambda i: (i, 0)),
        pl.BlockSpec((BN, D), lambda i: (i, 0)),
    ],
    out_shape=[
        jax.ShapeDtypeStruct((N, D), _f32),
        jax.ShapeDtypeStruct((N, D), _f32),
    ],
)


def _fin_body(acc_ref, h_ref, l0_ref, b0_ref, l1_ref, b1_ref, o_ref):
    t = acc_ref[0] + acc_ref[1] + h_ref[...]
    h = _l2norm_rows(t)
    a = lax.dot_general(h, l0_ref[...], _dims, preferred_element_type=_f32)
    a = jnp.maximum(a + b0_ref[...], 0.0)
    a = lax.dot_general(a, l1_ref[...], _dims, preferred_element_type=_f32)
    o_ref[...] = jnp.maximum(a + b1_ref[...], 0.0)


_tfin = pl.pallas_call(
    _fin_body,
    grid=(GRID,),
    in_specs=[
        pl.BlockSpec((NC, BN, D), lambda i: (0, i, 0)),
        pl.BlockSpec((BN, D), lambda i: (i, 0)),
        pl.BlockSpec((D, D), lambda i: (0, 0)),
        pl.BlockSpec((1, D), lambda i: (0, 0)),
        pl.BlockSpec((D, D), lambda i: (0, 0)),
        pl.BlockSpec((1, D), lambda i: (0, 0)),
    ],
    out_specs=pl.BlockSpec((BN, D), lambda i: (i, 0)),
    out_shape=jax.ShapeDtypeStruct((N, D), _f32),
)


def _head_body(p_ref, wp_ref, bp_ref, o_ref):
    m = p_ref[0] + p_ref[1]
    o = jnp.sum(m * wp_ref[...], axis=1, keepdims=True)
    o_ref[...] = o + bp_ref[0, 0]


_thead = pl.pallas_call(
    _head_body,
    out_shape=jax.ShapeDtypeStruct((G, 1), _f32),
)


def kernel(x, edge_index, edge_attr, batch, embd, gammas, waW, waB,
           linW, linB, wpW, wpB):
    src = edge_index[0]
    dst = edge_index[1]
    ea = edge_attr[:, 0]
    gp = jnp.pad(gammas[:, :, 0], ((0, 0), (0, VP - V))).reshape(-1)

    h0, ew0, ew1, ew2 = _prep(x, ea, dst, gp, embd)
    ews = (ew0, ew1, ew2)
    h = h0
    hx = _t0(h0, waW[0], waB[0].reshape(1, D))
    for l in range(3):
        acc = _gcn(hx, src, dst, ews[l])
        if l < 2:
            h, hx = _tmid(acc, h, waW[l + 1], waB[l + 1].reshape(1, D))
        else:
            hfin = _tfin(acc, h, linW[0], linB[0].reshape(1, D),
                         linW[1], linB[1].reshape(1, D))
    part = _pool(hfin, batch)
    props = _thead(part, wpW, wpB.reshape(1, 1))
    return props[:, 0]


# speedup vs baseline: 1.2297x; 1.0181x over previous
import functools

import jax
import jax.numpy as jnp
from jax import lax
from jax.experimental import pallas as pl
from jax.experimental.pallas import tpu as pltpu
from jax.experimental.pallas import tpu_sc as plsc

N = 10000
E = 320000
D = 128
V = 100
VP = 112
G = 400
NC = 2
NS = 16
NW = NC * NS
RPW = 312
RC = 104
EPW = E // NW
EC = 80
NCH = EPW // EC

_mesh = plsc.VectorSubcoreMesh(core_axis_name="c", subcore_axis_name="s")
_f32 = jnp.float32
_i32 = jnp.int32


EPB0 = 9984


def _prep_body(x_hbm, ea_hbm, dst_hbm, gp_hbm, embd_hbm,
               h0_hbm, ew0_hbm, ew1_hbm, ew2_hbm,
               embd_sh, gpv, gtab, xv0, xv1, xv2, hrows0, hrows1, hrows2,
               xv16, hrows16,
               didx_all, av_all, xdv_all, ewb0, ewb1, ewb2,
               sem0, sem1, sem2, sem3):
    c = lax.axis_index("c")
    s = lax.axis_index("s")
    w = c * NS + s
    ew_hbms = (ew0_hbm, ew1_hbm, ew2_hbm)
    ewbs = (ewb0, ewb1, ewb2)
    ebase = w * EPB0

    dd = pltpu.async_copy(dst_hbm.at[pl.ds(ebase, EPB0)], didx_all, sem0)
    de = pltpu.async_copy(ea_hbm.at[pl.ds(ebase, EPB0)], av_all, sem1)

    @pl.when(s == 0)
    def _():
        pltpu.sync_copy(embd_hbm, embd_sh)

    pltpu.sync_copy(gp_hbm, gpv)
    for i in range(3 * VP // 16):
        v = gpv[pl.ds(16 * i, 16)]
        gtab[pl.ds(16 * i, 16)] = 1.0 / (1.0 + jnp.exp(-v))
    plsc.subcore_barrier()

    rbase = w * RPW
    hrows = (hrows0, hrows1, hrows2)
    xvs = (xv0, xv1, xv2)
    xls = [pltpu.async_copy(x_hbm.at[pl.ds(rbase + j * RC, RC)], xvs[j],
                            sem2)
           for j in range(3)]
    for xl in xls:
        xl.wait()
    gds = [pltpu.async_copy(embd_sh.at[xvs[j]], hrows[j], sem2)
           for j in range(3)]
    for g in gds:
        g.wait()
    hds = [pltpu.async_copy(hrows[j], h0_hbm.at[pl.ds(rbase + j * RC, RC)],
                            sem2)
           for j in range(3)]
    for h in hds:
        h.wait()

    @pl.when(w == 0)
    def _():
        b = NW * RPW
        pltpu.sync_copy(x_hbm.at[pl.ds(b, 16)], xv16)
        pltpu.sync_copy(embd_sh.at[xv16], hrows16)
        pltpu.sync_copy(hrows16, h0_hbm.at[pl.ds(b, 16)])

    dd.wait()
    xds = [pltpu.async_copy(
        x_hbm.at[didx_all.at[pl.ds(j * 128, 128)]],
        xdv_all.at[pl.ds(j * 128, 128)], sem3)
        for j in range(EPB0 // 128)]
    for xd in xds:
        xd.wait()
    de.wait()

    def grp(g, _):
        a = av_all[pl.ds(g * 16, 16)]
        xi = xdv_all[pl.ds(g * 16, 16)]
        a2 = a * a
        for l in range(3):
            gl = plsc.load_gather(gtab, [xi + (l * VP)])
            ewbs[l][pl.ds(g * 16, 16)] = jnp.exp(-gl * a2)
        return 0

    lax.fori_loop(0, EPB0 // 16, grp, 0)
    eds = [pltpu.async_copy(ewbs[l], ew_hbms[l].at[pl.ds(ebase, EPB0)],
                            sem2)
           for l in range(3)]
    for e in eds:
        e.wait()

    @pl.when(w < 4)
    def _():
        bt = NW * EPB0 + w * 128
        pltpu.sync_copy(dst_hbm.at[pl.ds(bt, 128)],
                        didx_all.at[pl.ds(0, 128)])
        pltpu.sync_copy(ea_hbm.at[pl.ds(bt, 128)],
                        av_all.at[pl.ds(0, 128)])
        pltpu.sync_copy(x_hbm.at[didx_all.at[pl.ds(0, 128)]],
                        xdv_all.at[pl.ds(0, 128)])
        lax.fori_loop(0, 8, grp, 0)
        for l in range(3):
            pltpu.sync_copy(ewbs[l].at[pl.ds(0, 128)],
                            ew_hbms[l].at[pl.ds(bt, 128)])


_prep = pl.kernel(
    _prep_body,
    out_type=(jax.ShapeDtypeStruct((N, D), _f32),
              jax.ShapeDtypeStruct((E,), _f32),
              jax.ShapeDtypeStruct((E,), _f32),
              jax.ShapeDtypeStruct((E,), _f32)),
    mesh=_mesh,
    compiler_params=pltpu.CompilerParams(needs_layout_passes=False),
    scratch_types=[
        pltpu.VMEM_SHARED((V, D), _f32),
        pltpu.VMEM((3 * VP,), _f32),
        pltpu.VMEM((3 * VP,), _f32),
        pltpu.VMEM((RC,), _i32),
        pltpu.VMEM((RC,), _i32),
        pltpu.VMEM((RC,), _i32),
        pltpu.VMEM((RC, D), _f32),
        pltpu.VMEM((RC, D), _f32),
        pltpu.VMEM((RC, D), _f32),
        pltpu.VMEM((16,), _i32),
        pltpu.VMEM((16, D), _f32),
        pltpu.VMEM((EPB0,), _i32),
        pltpu.VMEM((EPB0,), _f32),
        pltpu.VMEM((EPB0,), _i32),
        pltpu.VMEM((EPB0,), _f32),
        pltpu.VMEM((EPB0,), _f32),
        pltpu.VMEM((EPB0,), _f32),
        pltpu.SemaphoreType.DMA,
        pltpu.SemaphoreType.DMA,
        pltpu.SemaphoreType.DMA,
        pltpu.SemaphoreType.DMA,
    ],
)


CB = 96
CPW = 104
EPB = CPW * CB
NSLOT = 4
TEPW = 32


def _scale_chunk(rows, ewc, nedge):
    def grp(g, _):
        for e in range(16):
            wb = plsc.load_gather(ewc, [jnp.zeros((16,), _i32)
                                        + (g * 16 + e)])
            i = g * 16 + e
            for k in range(D // 16):
                rows[i, pl.ds(16 * k, 16)] = (
                    rows[i, pl.ds(16 * k, 16)] * wb)
        return 0

    lax.fori_loop(0, nedge // 16, grp, 0)
    for e in range(nedge % 16):
        i = (nedge // 16) * 16 + e
        wb = plsc.load_gather(ewc, [jnp.zeros((16,), _i32) + i])
        for k in range(D // 16):
            rows[i, pl.ds(16 * k, 16)] = rows[i, pl.ds(16 * k, 16)] * wb


def _gcn_body(hx_hbm, src_hbm, dst_hbm, ew_hbm,
              out_hbm,
              acc_sh, rows0, rows1, rows2, rows3,
              sbuf0, sbuf1, sbuf2, sbuf3, dbuf0, dbuf1, dbuf2, dbuf3,
              ewc0, ewc1, ewc2, ewc3, tsb, tdb, tew,
              lsem0, lsem1, lsem2, lsem3, gsem0, gsem1, gsem2, gsem3,
              ssem0, ssem1, ssem2, ssem3, dsem0, dsem1, dsem2, dsem3):
    c = lax.axis_index("c")
    s = lax.axis_index("s")
    w = c * NS + s
    ebase = w * EPB
    rows = (rows0, rows1, rows2, rows3)
    sbuf = (sbuf0, sbuf1, sbuf2, sbuf3)
    dbuf = (dbuf0, dbuf1, dbuf2, dbuf3)
    ewc = (ewc0, ewc1, ewc2, ewc3)
    lsem = (lsem0, lsem1, lsem2, lsem3)
    gsem = (gsem0, gsem1, gsem2, gsem3)
    ssem = (ssem0, ssem1, ssem2, ssem3)
    dsem = (dsem0, dsem1, dsem2, dsem3)

    def zrow(i, _):
        for k in range(D // 16):
            rows0[i, pl.ds(16 * k, 16)] = jnp.zeros((16,), _f32)
        return 0

    lax.fori_loop(0, 8, zrow, 0)
    zsrc = rows0.at[pl.ds(0, 8)]
    zds = [pltpu.async_copy(zsrc, acc_sh.at[pl.ds(s * 624 + j * 8, 8)],
                            gsem0)
           for j in range(78)]
    for zd in zds:
        zd.wait()

    @pl.when(s == 0)
    def _():
        pltpu.sync_copy(zsrc, acc_sh.at[pl.ds(9984, 8)])
        pltpu.sync_copy(zsrc, acc_sh.at[pl.ds(9992, 8)])

    plsc.subcore_barrier()

    for t in range(NSLOT):
        pltpu.async_copy(src_hbm.at[pl.ds(ebase + t * CB, CB)], sbuf[t],
                         lsem[t])
        pltpu.async_copy(ew_hbm.at[pl.ds(ebase + t * CB, CB)], ewc[t],
                         lsem[t])

    def body(b, _):
        gds = []
        for t in range(NSLOT):
            off = (NSLOT * b + t) * CB
            @pl.when(b > 0)
            def _():
                pltpu.make_async_copy(rows[t], acc_sh.at[dbuf[t]],
                                      ssem[t]).wait()

            dl = pltpu.async_copy(dst_hbm.at[pl.ds(ebase + off, CB)],
                                  dbuf[t], dsem[t])
            pltpu.make_async_copy(src_hbm.at[pl.ds(ebase + off, CB)],
                                  sbuf[t], lsem[t]).wait()
            pltpu.make_async_copy(ew_hbm.at[pl.ds(ebase + off, CB)],
                                  ewc[t], lsem[t]).wait()
            gds.append((pltpu.async_copy(hx_hbm.at[sbuf[t]], rows[t],
                                         gsem[t]), dl))
        for t in range(NSLOT):
            off = (NSLOT * b + t) * CB
            g, dl = gds[t]
            g.wait()
            pltpu.async_copy(
                src_hbm.at[pl.ds(ebase + off + NSLOT * CB, CB)],
                sbuf[t], lsem[t])
            _scale_chunk(rows[t], ewc[t], CB)
            pltpu.async_copy(
                ew_hbm.at[pl.ds(ebase + off + NSLOT * CB, CB)],
                ewc[t], lsem[t])
            dl.wait()
            pltpu.async_copy(rows[t], acc_sh.at[dbuf[t]], ssem[t],
                             add=True)
        return 0

    lax.fori_loop(0, CPW // NSLOT, body, 0)
    for t in range(NSLOT):
        pltpu.make_async_copy(rows[t], acc_sh.at[dbuf[t]], ssem[t]).wait()
        pltpu.make_async_copy(src_hbm.at[pl.ds(ebase, CB)], sbuf[t],
                              lsem[t]).wait()
        pltpu.make_async_copy(ew_hbm.at[pl.ds(ebase, CB)], ewc[t],
                              lsem[t]).wait()

    bt = NW * EPB + w * TEPW

    @pl.when(w < 16)
    def _():
        pltpu.sync_copy(src_hbm.at[pl.ds(bt, TEPW)], tsb)
        pltpu.sync_copy(dst_hbm.at[pl.ds(bt, TEPW)], tdb)
        pltpu.sync_copy(ew_hbm.at[pl.ds(bt, TEPW)], tew)
        pltpu.sync_copy(hx_hbm.at[tsb], rows0.at[pl.ds(0, TEPW)])
        _scale_chunk(rows0, tew, TEPW)
        pltpu.sync_copy(rows0.at[pl.ds(0, TEPW)], acc_sh.at[tdb],
                        add=True)

    plsc.subcore_barrier()
    for j in range(6):
        o = s * 624 + j * RC
        pltpu.sync_copy(acc_sh.at[pl.ds(o, RC)],
                        out_hbm.at[c, pl.ds(o, RC)])

    @pl.when(s == 0)
    def _():
        pltpu.sync_copy(acc_sh.at[pl.ds(9984, 16)],
                        out_hbm.at[c, pl.ds(9984, 16)])


_gcn = pl.kernel(
    _gcn_body,
    out_type=jax.ShapeDtypeStruct((NC, N, D), _f32),
    mesh=_mesh,
    compiler_params=pltpu.CompilerParams(needs_layout_passes=False),
    scratch_types=(
        [pltpu.VMEM_SHARED((N, D), _f32)]
        + [pltpu.VMEM((CB, D), _f32)] * 4
        + [pltpu.VMEM((CB,), _i32)] * 8
        + [pltpu.VMEM((CB,), _f32)] * 4
        + [pltpu.VMEM((TEPW,), _i32)] * 2
        + [pltpu.VMEM((TEPW,), _f32)]
        + [pltpu.SemaphoreType.DMA] * 16
    ),
)


def _pool_body(h_hbm, batch_hbm, out_hbm,
               acc_sh, zbuf, vals, bidx, vals16, bidx16):
    c = lax.axis_index("c")
    s = lax.axis_index("s")
    w = c * NS + s

    def zrow(i, _):
        for k in range(D // 16):
            zbuf[i, pl.ds(16 * k, 16)] = jnp.zeros((16,), _f32)
        return 0

    lax.fori_loop(0, 24, zrow, 0)
    pltpu.sync_copy(zbuf, acc_sh.at[pl.ds(s * 24, 24)])

    @pl.when(s == 0)
    def _():
        pltpu.sync_copy(zbuf.at[pl.ds(0, 16)], acc_sh.at[pl.ds(384, 16)])

    plsc.subcore_barrier()

    rbase = w * RPW

    def chunk(j, _):
        b = rbase + j * RC
        pltpu.sync_copy(h_hbm.at[pl.ds(b, RC)], vals)
        pltpu.sync_copy(batch_hbm.at[pl.ds(b, RC)], bidx)
        pltpu.sync_copy(vals, acc_sh.at[bidx], add=True)
        return 0

    lax.fori_loop(0, RPW // RC, chunk, 0)

    @pl.when(w == 0)
    def _():
        b = NW * RPW
        pltpu.sync_copy(h_hbm.at[pl.ds(b, 16)], vals16)
        pltpu.sync_copy(batch_hbm.at[pl.ds(b, 16)], bidx16)
        pltpu.sync_copy(vals16, acc_sh.at[bidx16], add=True)

    plsc.subcore_barrier()
    pltpu.sync_copy(acc_sh.at[pl.ds(s * 24, 24)],
                    out_hbm.at[c, pl.ds(s * 24, 24)])

    @pl.when(s == 0)
    def _():
        pltpu.sync_copy(acc_sh.at[pl.ds(384, 16)],
                        out_hbm.at[c, pl.ds(384, 16)])


_pool = pl.kernel(
    _pool_body,
    out_type=jax.ShapeDtypeStruct((NC, G, D), _f32),
    mesh=_mesh,
    compiler_params=pltpu.CompilerParams(needs_layout_passes=False),
    scratch_types=[
        pltpu.VMEM_SHARED((G, D), _f32),
        pltpu.VMEM((24, D), _f32),
        pltpu.VMEM((RC, D), _f32),
        pltpu.VMEM((RC,), _i32),
        pltpu.VMEM((16, D), _f32),
        pltpu.VMEM((16,), _i32),
    ],
)


BN = 2000
GRID = N // BN
_dims = (((1,), (1,)), ((), ()))


def _t0_body(h_ref, w_ref, b_ref, o_ref):
    hx = lax.dot_general(h_ref[...], w_ref[...], _dims,
                         preferred_element_type=_f32)
    o_ref[...] = jnp.maximum(hx + b_ref[...], 0.0)


_t0 = pl.pallas_call(
    _t0_body,
    grid=(GRID,),
    in_specs=[
        pl.BlockSpec((BN, D), lambda i: (i, 0)),
        pl.BlockSpec((D, D), lambda i: (0, 0)),
        pl.BlockSpec((1, D), lambda i: (0, 0)),
    ],
    out_specs=pl.BlockSpec((BN, D), lambda i: (i, 0)),
    out_shape=jax.ShapeDtypeStruct((N, D), _f32),
)


def _l2norm_rows(t):
    n = jnp.sqrt(jnp.sum(t * t, axis=1, keepdims=True))
    return t / jnp.maximum(n, 1e-12)


def _mid_body(acc_ref, h_ref, w_ref, b_ref, hn_ref, hx_ref):
    t = acc_ref[0] + acc_ref[1] + h_ref[...]
    h = _l2norm_rows(t)
    hn_ref[...] = h
    hx = lax.dot_general(h, w_ref[...], _dims, preferred_element_type=_f32)
    hx_ref[...] = jnp.maximum(hx + b_ref[...], 0.0)


_tmid = pl.pallas_call(
    _mid_body,
    grid=(GRID,),
    in_specs=[
        pl.BlockSpec((NC, BN, D), lambda i: (0, i, 0)),
        pl.BlockSpec((BN, D), lambda i: (i, 0)),
        pl.BlockSpec((D, D), lambda i: (0, 0)),
        pl.BlockSpec((1, D), lambda i: (0, 0)),
    ],
    out_specs=[
        pl.BlockSpec((BN, D), lambda i: (i, 0)),
        pl.BlockSpec((BN, D), lambda i: (i, 0)),
    ],
    out_shape=[
        jax.ShapeDtypeStruct((N, D), _f32),
        jax.ShapeDtypeStruct((N, D), _f32),
    ],
)


def _fin_body(acc_ref, h_ref, l0_ref, b0_ref, l1_ref, b1_ref, o_ref):
    t = acc_ref[0] + acc_ref[1] + h_ref[...]
    h = _l2norm_rows(t)
    a = lax.dot_general(h, l0_ref[...], _dims, preferred_element_type=_f32)
    a = jnp.maximum(a + b0_ref[...], 0.0)
    a = lax.dot_general(a, l1_ref[...], _dims, preferred_element_type=_f32)
    o_ref[...] = jnp.maximum(a + b1_ref[...], 0.0)


_tfin = pl.pallas_call(
    _fin_body,
    grid=(GRID,),
    in_specs=[
        pl.BlockSpec((NC, BN, D), lambda i: (0, i, 0)),
        pl.BlockSpec((BN, D), lambda i: (i, 0)),
        pl.BlockSpec((D, D), lambda i: (0, 0)),
        pl.BlockSpec((1, D), lambda i: (0, 0)),
        pl.BlockSpec((D, D), lambda i: (0, 0)),
        pl.BlockSpec((1, D), lambda i: (0, 0)),
    ],
    out_specs=pl.BlockSpec((BN, D), lambda i: (i, 0)),
    out_shape=jax.ShapeDtypeStruct((N, D), _f32),
)


def _head_body(p_ref, wp_ref, bp_ref, o_ref):
    m = p_ref[0] + p_ref[1]
    o = jnp.sum(m * wp_ref[...], axis=1, keepdims=True)
    o_ref[...] = o + bp_ref[0, 0]


_thead = pl.pallas_call(
    _head_body,
    out_shape=jax.ShapeDtypeStruct((G, 1), _f32),
)


def kernel(x, edge_index, edge_attr, batch, embd, gammas, waW, waB,
           linW, linB, wpW, wpB):
    src = edge_index[0]
    dst = edge_index[1]
    ea = edge_attr[:, 0]
    gp = jnp.pad(gammas[:, :, 0], ((0, 0), (0, VP - V))).reshape(-1)

    h0, ew0, ew1, ew2 = _prep(x, ea, dst, gp, embd)
    ews = (ew0, ew1, ew2)
    h = h0
    hx = _t0(h0, waW[0], waB[0].reshape(1, D))
    for l in range(3):
        acc = _gcn(hx, src, dst, ews[l])
        if l < 2:
            h, hx = _tmid(acc, h, waW[l + 1], waB[l + 1].reshape(1, D))
        else:
            hfin = _tfin(acc, h, linW[0], linB[0].reshape(1, D),
                         linW[1], linB[1].reshape(1, D))
    part = _pool(hfin, batch)
    props = _thead(part, wpW, wpB.reshape(1, 1))
    return props[:, 0]
